# baseline (speedup 1.0000x reference)
import functools

import jax
import jax.numpy as jnp
from jax import lax
from jax.experimental import pallas as pl
from jax.experimental.pallas import tpu as pltpu
from jax.experimental.pallas import tpu_sc as plsc

HIDDEN = 768
L = 16
NSLICES = HIDDEN // L
NC, NS = 2, 16
NW = NC * NS
LN_EPS = 1e-12


def _rsqrt(x):
    i = lax.bitcast_convert_type(x, jnp.int32)
    i = jnp.int32(0x5F3759DF) - (i >> 1)
    y = lax.bitcast_convert_type(i, jnp.float32)
    for _ in range(3):
        y = y * (1.5 - 0.5 * x * y * y)
    return y


def kernel(input_ids, token_type_ids, word_emb, pos_emb, type_emb, gamma, beta):
    B, S = input_ids.shape
    N = B * S
    per_w = N // NW
    K = 16
    NBUF = 4
    nchunks = per_w // K

    ids = input_ids.reshape(N).astype(jnp.int32)
    tts = token_type_ids.reshape(N).astype(jnp.int32)
    pos_ids = jnp.tile(jnp.arange(S, dtype=jnp.int32), B)

    mesh = plsc.VectorSubcoreMesh(core_axis_name="c", subcore_axis_name="s")

    @functools.partial(
        pl.kernel,
        out_type=jax.ShapeDtypeStruct((N, HIDDEN), jnp.float32),
        mesh=mesh,
        compiler_params=pltpu.CompilerParams(needs_layout_passes=False),
        scratch_types=[
            pltpu.VMEM((per_w,), jnp.int32),
            pltpu.VMEM((per_w,), jnp.int32),
            pltpu.VMEM((per_w + L,), jnp.int32),
            pltpu.VMEM((NBUF, K, HIDDEN), jnp.float32),
            pltpu.VMEM((NBUF, K, HIDDEN), jnp.float32),
            pltpu.VMEM((2, HIDDEN), jnp.float32),
            pltpu.VMEM((HIDDEN,), jnp.float32),
            pltpu.VMEM((HIDDEN,), jnp.float32),
            pltpu.SemaphoreType.DMA,
            pltpu.SemaphoreType.DMA,
            pltpu.SemaphoreType.DMA,
        ],
    )
    def run(ids_hbm, pids_hbm, tt_hbm, word_hbm, pos_hbm, type_hbm,
            gamma_hbm, beta_hbm, out_hbm,
            idx_v, pidx_v, tt_v, rows_v, prow_v, type_v, g_v, b_v,
            sem_w, sem_p, sem_o):
        wid = lax.axis_index("s") * NC + lax.axis_index("c")
        base = wid * per_w

        pltpu.sync_copy(ids_hbm.at[pl.ds(base, per_w)], idx_v)
        pltpu.sync_copy(pids_hbm.at[pl.ds(base, per_w)], pidx_v)
        pltpu.sync_copy(tt_hbm.at[pl.ds(base, per_w)],
                        tt_v.at[pl.ds(0, per_w)])
        pltpu.sync_copy(type_hbm, type_v)
        pltpu.sync_copy(gamma_hbm, g_v)
        pltpu.sync_copy(beta_hbm, b_v)

        def start(c, b):
            pltpu.async_copy(
                word_hbm.at[idx_v.at[pl.ds(c * K, K)]], rows_v.at[b], sem_w)
            pltpu.async_copy(
                pos_hbm.at[pidx_v.at[pl.ds(c * K, K)]], prow_v.at[b], sem_p)

        def wait_gather(b):
            pltpu.make_async_copy(
                word_hbm.at[pl.ds(0, K)], rows_v.at[b], sem_w).wait()
            pltpu.make_async_copy(
                pos_hbm.at[pl.ds(0, K)], prow_v.at[b], sem_p).wait()

        def wait_out(b):
            pltpu.make_async_copy(
                rows_v.at[b], out_hbm.at[pl.ds(base, K)], sem_o).wait()

        T = 16

        def compute(cbase, b):
            @plsc.parallel_loop(0, K, step=T)
            def _(r0):
                masks = []
                for t in range(T):
                    tti = tt_v[pl.ds(cbase + r0 + t, L)][0]
                    masks.append(jnp.full((L,), tti, jnp.int32) == 1)
                zero = jnp.zeros((L,), jnp.float32)

                def pass_a(j, carry):
                    accs = list(carry[0])
                    accq = list(carry[1])
                    sl = pl.ds(j * L, L)
                    t0 = type_v[0, sl]
                    t1 = type_v[1, sl]
                    for t in range(T):
                        v = (rows_v[b, r0 + t, sl] + prow_v[b, r0 + t, sl]
                             + jnp.where(masks[t], t1, t0))
                        rows_v[b, r0 + t, sl] = v
                        accs[t] = accs[t] + v
                        accq[t] = accq[t] + v * v
                    return tuple(accs), tuple(accq)

                init = (tuple([zero] * T), tuple([zero] * T))
                accs, accq = plsc.parallel_loop(
                    0, NSLICES, unroll=2, carry=init)(pass_a)

                meanvs, rstds = [], []
                for t in range(T):
                    mean = jnp.sum(accs[t]) * (1.0 / HIDDEN)
                    meansq = jnp.sum(accq[t]) * (1.0 / HIDDEN)
                    var = meansq - mean * mean
                    rstds.append(
                        _rsqrt(jnp.full((L,), var + LN_EPS, jnp.float32)))
                    meanvs.append(jnp.full((L,), mean, jnp.float32))

                @plsc.parallel_loop(0, NSLICES, unroll=2)
                def pass_b(j):
                    sl = pl.ds(j * L, L)
                    gj = g_v[sl]
                    bj = b_v[sl]
                    for t in range(T):
                        v = (rows_v[b, r0 + t, sl] - meanvs[t]) * rstds[t]
                        rows_v[b, r0 + t, sl] = v * gj + bj

        start(0, 0)
        start(1, 1)

        @pl.loop(0, nchunks, step=NBUF)
        def _(c):
            for b in range(NBUF):
                cc = c + b

                @pl.when(cc + 2 < nchunks)
                def _():
                    @pl.when(cc >= 2)
                    def _():
                        wait_out((b + 2) % NBUF)

                    start(cc + 2, (b + 2) % NBUF)

                wait_gather(b)
                compute(cc * K, b)
                pltpu.async_copy(
                    rows_v.at[b], out_hbm.at[pl.ds(base + cc * K, K)], sem_o)

        for b in range(NBUF):
            wait_out(b)

    out = run(ids, pos_ids, tts, word_emb, pos_emb, type_emb, gamma, beta)
    return out.reshape(B, S, HIDDEN)


# speedup vs baseline: 1.8148x; 1.8148x over previous
import functools

import jax
import jax.numpy as jnp
from jax import lax
from jax.experimental import pallas as pl
from jax.experimental.pallas import tpu as pltpu
from jax.experimental.pallas import tpu_sc as plsc

HIDDEN = 768
L = 16
NSLICES = HIDDEN // L
NC, NS = 2, 16
NW = NC * NS
LN_EPS = 1e-12


def _rsqrt(x):
    i = lax.bitcast_convert_type(x, jnp.int32)
    i = jnp.int32(0x5F3759DF) - (i >> 1)
    y = lax.bitcast_convert_type(i, jnp.float32)
    for _ in range(3):
        y = y * (1.5 - 0.5 * x * y * y)
    return y


def kernel(input_ids, token_type_ids, word_emb, pos_emb, type_emb, gamma, beta):
    B, S = input_ids.shape
    N = B * S
    per_w = N // NW
    K = 16
    NBUF = 4
    nchunks = per_w // K

    ids = input_ids.reshape(N).astype(jnp.int32)
    tts = token_type_ids.reshape(N).astype(jnp.int32)
    pos_ids = jnp.tile(jnp.arange(S, dtype=jnp.int32), B)

    mesh = plsc.VectorSubcoreMesh(core_axis_name="c", subcore_axis_name="s")

    @functools.partial(
        pl.kernel,
        out_type=jax.ShapeDtypeStruct((N, HIDDEN), jnp.float32),
        mesh=mesh,
        compiler_params=pltpu.CompilerParams(needs_layout_passes=False),
        scratch_types=[
            pltpu.VMEM((per_w,), jnp.int32),
            pltpu.VMEM((per_w,), jnp.int32),
            pltpu.VMEM((per_w + L,), jnp.int32),
            pltpu.VMEM((NBUF, K, HIDDEN), jnp.float32),
            pltpu.VMEM((NBUF, K, HIDDEN), jnp.float32),
            pltpu.VMEM((2, HIDDEN), jnp.float32),
            pltpu.VMEM((HIDDEN,), jnp.float32),
            pltpu.VMEM((HIDDEN,), jnp.float32),
            pltpu.SemaphoreType.DMA,
            pltpu.SemaphoreType.DMA,
            pltpu.SemaphoreType.DMA,
        ],
    )
    def run(ids_hbm, pids_hbm, tt_hbm, word_hbm, pos_hbm, type_hbm,
            gamma_hbm, beta_hbm, out_hbm,
            idx_v, pidx_v, tt_v, rows_v, prow_v, type_v, g_v, b_v,
            sem_w, sem_p, sem_o):
        wid = lax.axis_index("s") * NC + lax.axis_index("c")
        base = wid * per_w

        d1 = pltpu.async_copy(ids_hbm.at[pl.ds(base, per_w)], idx_v, sem_o)
        d2 = pltpu.async_copy(pids_hbm.at[pl.ds(base, per_w)], pidx_v, sem_o)
        d3 = pltpu.async_copy(tt_hbm.at[pl.ds(base, per_w)],
                              tt_v.at[pl.ds(0, per_w)], sem_p)
        d4 = pltpu.async_copy(type_hbm, type_v, sem_p)
        d5 = pltpu.async_copy(gamma_hbm, g_v, sem_p)
        d6 = pltpu.async_copy(beta_hbm, b_v, sem_p)
        d1.wait()
        d2.wait()

        def start(c, b):
            pltpu.async_copy(
                word_hbm.at[idx_v.at[pl.ds(c * K, K)]], rows_v.at[b], sem_w)
            pltpu.async_copy(
                pos_hbm.at[pidx_v.at[pl.ds(c * K, K)]], prow_v.at[b], sem_p)

        def wait_gather(b):
            pltpu.make_async_copy(
                word_hbm.at[pl.ds(0, K)], rows_v.at[b], sem_w).wait()
            pltpu.make_async_copy(
                pos_hbm.at[pl.ds(0, K)], prow_v.at[b], sem_p).wait()

        def wait_out(b):
            pltpu.make_async_copy(
                rows_v.at[b], out_hbm.at[pl.ds(base, K)], sem_o).wait()

        T = 8

        def compute(cbase, b):
            @plsc.parallel_loop(0, K, step=T)
            def _(r0):
                masks = []
                for t in range(T):
                    tti = tt_v[pl.ds(cbase + r0 + t, L)][0]
                    masks.append(jnp.full((L,), tti, jnp.int32) == 1)
                zero = jnp.zeros((L,), jnp.float32)

                def pass_a(j, carry):
                    accs = list(carry[0])
                    accq = list(carry[1])
                    sl = pl.ds(j * L, L)
                    t0 = type_v[0, sl]
                    t1 = type_v[1, sl]
                    for t in range(T):
                        v = (rows_v[b, r0 + t, sl] + prow_v[b, r0 + t, sl]
                             + jnp.where(masks[t], t1, t0))
                        rows_v[b, r0 + t, sl] = v
                        accs[t] = accs[t] + v
                        accq[t] = accq[t] + v * v
                    return tuple(accs), tuple(accq)

                init = (tuple([zero] * T), tuple([zero] * T))
                accs, accq = plsc.parallel_loop(
                    0, NSLICES, unroll=2, carry=init)(pass_a)

                meanvs, rstds = [], []
                for t in range(T):
                    mean = jnp.sum(accs[t]) * (1.0 / HIDDEN)
                    meansq = jnp.sum(accq[t]) * (1.0 / HIDDEN)
                    var = meansq - mean * mean
                    rstds.append(
                        _rsqrt(jnp.full((L,), var + LN_EPS, jnp.float32)))
                    meanvs.append(jnp.full((L,), mean, jnp.float32))

                @plsc.parallel_loop(0, NSLICES, unroll=2)
                def pass_b(j):
                    sl = pl.ds(j * L, L)
                    gj = g_v[sl]
                    bj = b_v[sl]
                    for t in range(T):
                        v = (rows_v[b, r0 + t, sl] - meanvs[t]) * rstds[t]
                        rows_v[b, r0 + t, sl] = v * gj + bj

        start(0, 0)
        start(1, 1)
        d3.wait()
        d4.wait()
        d5.wait()
        d6.wait()

        @pl.loop(0, nchunks, step=NBUF)
        def _(c):
            for b in range(NBUF):
                cc = c + b

                @pl.when(cc + 2 < nchunks)
                def _():
                    @pl.when(cc >= 2)
                    def _():
                        wait_out((b + 2) % NBUF)

                    start(cc + 2, (b + 2) % NBUF)

                wait_gather(b)
                compute(cc * K, b)
                pltpu.async_copy(
                    rows_v.at[b], out_hbm.at[pl.ds(base + cc * K, K)], sem_o)

        for b in range(NBUF):
            wait_out(b)

    out = run(ids, pos_ids, tts, word_emb, pos_emb, type_emb, gamma, beta)
    return out.reshape(B, S, HIDDEN)
